# baseline (speedup 1.0000x reference)
import functools

import jax
import jax.numpy as jnp
from jax import lax
from jax.experimental import pallas as pl
from jax.experimental.pallas import tpu as pltpu
from jax.experimental.pallas import tpu_sc as plsc

_B, _L = 4096, 50
_N = _B * _L
_D = 64
_NW = 32
_PER_W = _N // _NW
_CHUNK = 128
_NCH = _PER_W // _CHUNK


def _sc_gather(act_table, res_table, aidx, ridx):
    mesh = plsc.VectorSubcoreMesh(core_axis_name="c", subcore_axis_name="s")

    @functools.partial(
        pl.kernel,
        mesh=mesh,
        out_type=[
            jax.ShapeDtypeStruct((_N, _D), jnp.float32),
            jax.ShapeDtypeStruct((_N, _D), jnp.float32),
        ],
        scratch_types=[
            pltpu.VMEM((_NCH, _CHUNK), jnp.int32),
            pltpu.VMEM((_NCH, _CHUNK), jnp.int32),
            pltpu.VMEM((_CHUNK, _D), jnp.float32),
            pltpu.VMEM((_CHUNK, _D), jnp.float32),
            pltpu.SemaphoreType.DMA,
            pltpu.SemaphoreType.DMA,
        ],
        compiler_params=pltpu.CompilerParams(use_tc_tiling_on_sc=False),
    )
    def k(act_hbm, res_hbm, aidx_hbm, ridx_hbm, out_a, out_r,
          aidx_v, ridx_v, rows_a, rows_r, sem_a, sem_r):
        wid = lax.axis_index("s") * 2 + lax.axis_index("c")
        pltpu.sync_copy(aidx_hbm.at[wid], aidx_v)
        pltpu.sync_copy(ridx_hbm.at[wid], ridx_v)
        base = wid * _PER_W

        def body(j, carry):
            off = pl.multiple_of(base + j * _CHUNK, _CHUNK)
            ca = pltpu.async_copy(act_hbm.at[aidx_v.at[j]], rows_a, sem_a)
            cr = pltpu.async_copy(res_hbm.at[ridx_v.at[j]], rows_r, sem_r)
            ca.wait()
            pltpu.sync_copy(rows_a, out_a.at[pl.ds(off, _CHUNK)])
            cr.wait()
            pltpu.sync_copy(rows_r, out_r.at[pl.ds(off, _CHUNK)])
            return carry

        lax.fori_loop(0, _NCH, body, 0)

    return k(act_table, res_table, aidx, ridx)


_NP = _N // 2


def _dense_body(np_, tp, w1n, b1n, w1t, b1t, qn, qt, bias, d_ref):
    f32 = jnp.float32
    dot = lambda x, w: jnp.dot(x, w, preferred_element_type=f32)
    hn = jnp.maximum(dot(np_[...], w1n[...]) + b1n[...], 0.0)
    ht = jnp.maximum(dot(tp[...], w1t[...]) + b1t[...], 0.0)
    d_ref[...] = dot(hn, qn[...]) + dot(ht, qt[...]) + bias[...]


def _tc_mlp(np_, tp, w1n, b1n, w1t, b1t, qn2, qt2, bias2):
    blk = 1024
    grid = (_NP // blk,)
    full = lambda i: (0, 0)
    tok = lambda i: (i, 0)
    return pl.pallas_call(
        _dense_body,
        grid=grid,
        in_specs=[
            pl.BlockSpec((blk, 32), tok),
            pl.BlockSpec((blk, 8), tok),
            pl.BlockSpec((32, 128), full),
            pl.BlockSpec((1, 128), full),
            pl.BlockSpec((8, 128), full),
            pl.BlockSpec((1, 128), full),
            pl.BlockSpec((128, 128), full),
            pl.BlockSpec((128, 128), full),
            pl.BlockSpec((1, 128), full),
        ],
        out_specs=pl.BlockSpec((blk, 128), tok),
        out_shape=jax.ShapeDtypeStruct((_NP, 128), jnp.float32),
    )(np_, tp, w1n, b1n, w1t, b1t, qn2, qt2, bias2)


def _combine_body(ap, rp, d, pa, pr, out_ref):
    f32 = jnp.float32
    dot = lambda x, w: jnp.dot(x, w, preferred_element_type=f32)
    out_ref[...] = dot(ap[...], pa[...]) + dot(rp[...], pr[...]) + d[...]


def _tc_combine(ap, rp, d, pa2, pr2):
    blk = 1024
    grid = (_NP // blk,)
    full = lambda i: (0, 0)
    tok = lambda i: (i, 0)
    return pl.pallas_call(
        _combine_body,
        grid=grid,
        in_specs=[
            pl.BlockSpec((blk, 128), tok),
            pl.BlockSpec((blk, 128), tok),
            pl.BlockSpec((blk, 128), tok),
            pl.BlockSpec((128, 128), full),
            pl.BlockSpec((128, 128), full),
        ],
        out_specs=pl.BlockSpec((blk, 128), tok),
        out_shape=jax.ShapeDtypeStruct((_NP, 128), jnp.float32),
    )(ap, rp, d, pa2, pr2)


def _bdiag(w):
    k, d = w.shape
    z = jnp.zeros((k, d), w.dtype)
    return jnp.concatenate(
        [jnp.concatenate([w, z], axis=1), jnp.concatenate([z, w], axis=1)], axis=0)


def kernel(act_ids, res_ids, num_feats, time_feats, act_table, res_table,
           num_W1, num_b1, num_W2, num_b2,
           time_W1, time_b1, time_W2, time_b2, proj_W, proj_b):
    aidx = act_ids.reshape(_NW, _NCH, _CHUNK).astype(jnp.int32)
    ridx = res_ids.reshape(_NW, _NCH, _CHUNK).astype(jnp.int32)
    a, r = _sc_gather(act_table, res_table, aidx, ridx)

    ap = a.reshape(_NP, 128)
    rp = r.reshape(_NP, 128)
    np_ = num_feats.reshape(_NP, 32)
    tp = jnp.pad(time_feats.reshape(_NP, 6), ((0, 0), (0, 2)))

    pa_s, pr_s = proj_W[0:64], proj_W[64:128]
    pn_s, pt_s = proj_W[128:192], proj_W[192:256]
    qn = num_W2 @ pn_s
    qt = time_W2 @ pt_s
    bias = num_b2 @ pn_s + time_b2 @ pt_s + proj_b
    w1t2 = jnp.pad(_bdiag(time_W1), ((0, 2), (0, 0)))

    two = lambda b: jnp.concatenate([b, b]).reshape(1, 128)
    d = _tc_mlp(np_, tp, _bdiag(num_W1), two(num_b1), w1t2, two(time_b1),
                _bdiag(qn), _bdiag(qt), two(bias))
    out2 = _tc_combine(ap, rp, d, _bdiag(pa_s), _bdiag(pr_s))
    return out2.reshape(_B, _L, _D)


# speedup vs baseline: 1.1266x; 1.1266x over previous
import functools

import jax
import jax.numpy as jnp
from jax import lax
from jax.experimental import pallas as pl
from jax.experimental.pallas import tpu as pltpu
from jax.experimental.pallas import tpu_sc as plsc

_B, _L = 4096, 50
_N = _B * _L
_D = 64
_NW = 32
_PER_W = _N // _NW
_CHUNK = 128
_NCH = _PER_W // _CHUNK


def _sc_gather(act_table, res_table, aidx, ridx):
    mesh = plsc.VectorSubcoreMesh(core_axis_name="c", subcore_axis_name="s")

    @functools.partial(
        pl.kernel,
        mesh=mesh,
        out_type=[
            jax.ShapeDtypeStruct((_N, _D), jnp.float32),
            jax.ShapeDtypeStruct((_N, _D), jnp.float32),
        ],
        scratch_types=[
            pltpu.VMEM((_NCH, _CHUNK), jnp.int32),
            pltpu.VMEM((_NCH, _CHUNK), jnp.int32),
            pltpu.VMEM((_CHUNK, _D), jnp.float32),
            pltpu.VMEM((_CHUNK, _D), jnp.float32),
            pltpu.SemaphoreType.DMA,
            pltpu.SemaphoreType.DMA,
        ],
        compiler_params=pltpu.CompilerParams(use_tc_tiling_on_sc=False),
    )
    def k(act_hbm, res_hbm, aidx_hbm, ridx_hbm, out_a, out_r,
          aidx_v, ridx_v, rows_a, rows_r, sem_a, sem_r):
        wid = lax.axis_index("s") * 2 + lax.axis_index("c")
        pltpu.sync_copy(aidx_hbm.at[wid], aidx_v)
        pltpu.sync_copy(ridx_hbm.at[wid], ridx_v)
        base = wid * _PER_W

        def body(j, carry):
            off = pl.multiple_of(base + j * _CHUNK, _CHUNK)
            ca = pltpu.async_copy(act_hbm.at[aidx_v.at[j]], rows_a, sem_a)
            cr = pltpu.async_copy(res_hbm.at[ridx_v.at[j]], rows_r, sem_r)
            ca.wait()
            pltpu.sync_copy(rows_a, out_a.at[pl.ds(off, _CHUNK)])
            cr.wait()
            pltpu.sync_copy(rows_r, out_r.at[pl.ds(off, _CHUNK)])
            return carry

        lax.fori_loop(0, _NCH, body, 0)

    return k(act_table, res_table, aidx, ridx)


_BTD = 256
_BTC = 512


def _dense_body(nf_ref, tf_ref, w1n, b1n, w1t, b1t, qn, qt, bias, d_ref):
    f32 = jnp.float32
    dot = lambda x, w: jnp.dot(x, w, preferred_element_type=f32)
    nf = nf_ref[...]
    tf = tf_ref[...]
    w1n_v, b1n_v = w1n[...], b1n[...]
    w1t_v, b1t_v = w1t[...], b1t[...]
    qn_v, qt_v, bias_v = qn[...], qt[...], bias[...]
    for l in range(_L):
        hn = jnp.maximum(dot(w1n_v, nf[l]) + b1n_v, 0.0)
        ht = jnp.maximum(dot(w1t_v, tf[:, l, :]) + b1t_v, 0.0)
        d_ref[l] = dot(qn_v, hn) + dot(qt_v, ht) + bias_v


def _tc_dense(nfT, tfT, w1nT, b1n, w1tT, b1t, qnT, qtT, bias):
    grid = (_B // _BTD,)
    full = lambda i: (0, 0)
    return pl.pallas_call(
        _dense_body,
        grid=grid,
        in_specs=[
            pl.BlockSpec((_L, 16, _BTD), lambda i: (0, 0, i)),
            pl.BlockSpec((3, _L, _BTD), lambda i: (0, 0, i)),
            pl.BlockSpec((_D, 16), full),
            pl.BlockSpec((_D, 1), full),
            pl.BlockSpec((_D, 3), full),
            pl.BlockSpec((_D, 1), full),
            pl.BlockSpec((_D, _D), full),
            pl.BlockSpec((_D, _D), full),
            pl.BlockSpec((_D, 1), full),
        ],
        out_specs=pl.BlockSpec((_L, _D, _BTD), lambda i: (0, 0, i)),
        out_shape=jax.ShapeDtypeStruct((_L, _D, _B), jnp.float32),
    )(nfT, tfT, w1nT, b1n, w1tT, b1t, qnT, qtT, bias)


def _combine_body(a_ref, r_ref, d_ref, pa, pr, out_ref):
    f32 = jnp.float32
    dn = (((0,), (1,)), ((), ()))
    ca = lax.dot_general(pa[...], a_ref[...], dn, preferred_element_type=f32)
    cr = lax.dot_general(pr[...], r_ref[...], dn, preferred_element_type=f32)
    out_ref[0] = d_ref[0] + ca + cr


def _tc_combine(a, r, d, pa_s, pr_s):
    nj = _B // _BTC
    grid = (_L, nj)
    full = lambda l, j: (0, 0)
    return pl.pallas_call(
        _combine_body,
        grid=grid,
        in_specs=[
            pl.BlockSpec((_BTC, _D), lambda l, j: (l * nj + j, 0)),
            pl.BlockSpec((_BTC, _D), lambda l, j: (l * nj + j, 0)),
            pl.BlockSpec((1, _D, _BTC), lambda l, j: (l, 0, j)),
            pl.BlockSpec((_D, _D), full),
            pl.BlockSpec((_D, _D), full),
        ],
        out_specs=pl.BlockSpec((1, _D, _BTC), lambda l, j: (l, 0, j)),
        out_shape=jax.ShapeDtypeStruct((_L, _D, _B), jnp.float32),
    )(a, r, d, pa_s, pr_s)


def kernel(act_ids, res_ids, num_feats, time_feats, act_table, res_table,
           num_W1, num_b1, num_W2, num_b2,
           time_W1, time_b1, time_W2, time_b2, proj_W, proj_b):
    aidx = jnp.transpose(act_ids).reshape(_NW, _NCH, _CHUNK).astype(jnp.int32)
    ridx = jnp.transpose(res_ids).reshape(_NW, _NCH, _CHUNK).astype(jnp.int32)
    a, r = _sc_gather(act_table, res_table, aidx, ridx)

    nfT = jnp.transpose(num_feats, (1, 2, 0))
    tfT = jnp.transpose(time_feats, (2, 1, 0))

    pa_s, pr_s = proj_W[0:64], proj_W[64:128]
    pn_s, pt_s = proj_W[128:192], proj_W[192:256]
    qnT = (num_W2 @ pn_s).T
    qtT = (time_W2 @ pt_s).T
    bias = (num_b2 @ pn_s + time_b2 @ pt_s + proj_b).reshape(_D, 1)

    d = _tc_dense(nfT, tfT, num_W1.T, num_b1.reshape(_D, 1),
                  time_W1.T, time_b1.reshape(_D, 1), qnT, qtT, bias)
    outT = _tc_combine(a, r, d, pa_s, pr_s)
    return jnp.transpose(outT, (2, 0, 1))


# speedup vs baseline: 1.7797x; 1.5797x over previous
import functools

import jax
import jax.numpy as jnp
from jax import lax
from jax.experimental import pallas as pl
from jax.experimental.pallas import tpu as pltpu
from jax.experimental.pallas import tpu_sc as plsc

_B, _L = 4096, 50
_N = _B * _L
_D = 64
_NW = 32
_PER_W = _N // _NW
_CHUNK = 128
_NCH = _PER_W // _CHUNK


def _sc_gather(act_table, res_table, aidx, ridx):
    mesh = plsc.VectorSubcoreMesh(core_axis_name="c", subcore_axis_name="s")

    @functools.partial(
        pl.kernel,
        mesh=mesh,
        out_type=[
            jax.ShapeDtypeStruct((_N, _D), jnp.float32),
            jax.ShapeDtypeStruct((_N, _D), jnp.float32),
        ],
        scratch_types=[
            pltpu.VMEM((_NCH, _CHUNK), jnp.int32),
            pltpu.VMEM((_NCH, _CHUNK), jnp.int32),
            pltpu.VMEM((_CHUNK, _D), jnp.float32),
            pltpu.VMEM((_CHUNK, _D), jnp.float32),
            pltpu.SemaphoreType.DMA,
            pltpu.SemaphoreType.DMA,
        ],
        compiler_params=pltpu.CompilerParams(use_tc_tiling_on_sc=False),
    )
    def k(act_hbm, res_hbm, aidx_hbm, ridx_hbm, out_a, out_r,
          aidx_v, ridx_v, rows_a, rows_r, sem_a, sem_r):
        wid = lax.axis_index("s") * 2 + lax.axis_index("c")
        pltpu.sync_copy(aidx_hbm.at[wid], aidx_v)
        pltpu.sync_copy(ridx_hbm.at[wid], ridx_v)
        base = wid * _PER_W

        def body(j, carry):
            off = pl.multiple_of(base + j * _CHUNK, _CHUNK)
            ca = pltpu.async_copy(act_hbm.at[aidx_v.at[j]], rows_a, sem_a)
            cr = pltpu.async_copy(res_hbm.at[ridx_v.at[j]], rows_r, sem_r)
            ca.wait()
            pltpu.sync_copy(rows_a, out_a.at[pl.ds(off, _CHUNK)])
            cr.wait()
            pltpu.sync_copy(rows_r, out_r.at[pl.ds(off, _CHUNK)])
            return carry

        lax.fori_loop(0, _NCH, body, 0)

    return k(act_table, res_table, aidx, ridx)


_BTD = 256
_BTC = 512


def _dense_body(nf_ref, tf_ref, w1n, b1n, w1t, b1t, qn, qt, bias, d_ref):
    f32 = jnp.float32
    dot = lambda x, w: jnp.dot(x, w, preferred_element_type=f32)
    nf = nf_ref[...]
    tf = tf_ref[...]
    w1n_v, b1n_v = w1n[...], b1n[...]
    w1t_v, b1t_v = w1t[...], b1t[...]
    qn_v, qt_v, bias_v = qn[...], qt[...], bias[...]
    for l in range(_L):
        hn = jnp.maximum(dot(w1n_v, nf[l]) + b1n_v, 0.0)
        ht = jnp.maximum(dot(w1t_v, tf[:, l, :]) + b1t_v, 0.0)
        d_ref[l] = dot(qn_v, hn) + dot(qt_v, ht) + bias_v


def _tc_dense(nfT, tfT, w1nT, b1n, w1tT, b1t, qnT, qtT, bias):
    grid = (_B // _BTD,)
    full = lambda i: (0, 0)
    return pl.pallas_call(
        _dense_body,
        grid=grid,
        in_specs=[
            pl.BlockSpec((_L, 16, _BTD), lambda i: (0, 0, i)),
            pl.BlockSpec((3, _L, _BTD), lambda i: (0, 0, i)),
            pl.BlockSpec((_D, 16), full),
            pl.BlockSpec((_D, 1), full),
            pl.BlockSpec((_D, 3), full),
            pl.BlockSpec((_D, 1), full),
            pl.BlockSpec((_D, _D), full),
            pl.BlockSpec((_D, _D), full),
            pl.BlockSpec((_D, 1), full),
        ],
        out_specs=pl.BlockSpec((_L, _D, _BTD), lambda i: (0, 0, i)),
        out_shape=jax.ShapeDtypeStruct((_L, _D, _B), jnp.float32),
    )(nfT, tfT, w1nT, b1n, w1tT, b1t, qnT, qtT, bias)


_H = _B // 2


def _combine_body(ap_ref, rp_ref, d_ref, pa, pr, out_ref):
    f32 = jnp.float32
    dn = (((0,), (1,)), ((), ()))
    ap_v, rp_v, pa_v, pr_v = ap_ref[...], rp_ref[...], pa[...], pr[...]
    left = (lax.dot_general(pa_v, ap_v[:, 0:_D], dn, preferred_element_type=f32)
            + lax.dot_general(pr_v, rp_v[:, 0:_D], dn, preferred_element_type=f32))
    right = (lax.dot_general(pa_v, ap_v[:, _D:128], dn, preferred_element_type=f32)
             + lax.dot_general(pr_v, rp_v[:, _D:128], dn, preferred_element_type=f32))
    out_ref[0] = d_ref[0] + jnp.concatenate([left, right], axis=1)


def _tc_combine(ap, rp, d, pa_s, pr_s):
    grid = (_L,)
    full = lambda l: (0, 0)
    return pl.pallas_call(
        _combine_body,
        grid=grid,
        in_specs=[
            pl.BlockSpec((_H, 128), lambda l: (l, 0)),
            pl.BlockSpec((_H, 128), lambda l: (l, 0)),
            pl.BlockSpec((1, _D, _B), lambda l: (l, 0, 0)),
            pl.BlockSpec((_D, _D), full),
            pl.BlockSpec((_D, _D), full),
        ],
        out_specs=pl.BlockSpec((1, _D, _B), lambda l: (l, 0, 0)),
        out_shape=jax.ShapeDtypeStruct((_L, _D, _B), jnp.float32),
    )(ap, rp, d, pa_s, pr_s)


def kernel(act_ids, res_ids, num_feats, time_feats, act_table, res_table,
           num_W1, num_b1, num_W2, num_b2,
           time_W1, time_b1, time_W2, time_b2, proj_W, proj_b):
    def _il(ids):
        t = jnp.transpose(ids).reshape(_L, 2, _H)
        return (jnp.transpose(t, (0, 2, 1)).reshape(_NW, _NCH, _CHUNK)
                .astype(jnp.int32))

    a, r = _sc_gather(act_table, res_table, _il(act_ids), _il(res_ids))

    nfT = jnp.transpose(num_feats, (1, 2, 0))
    tfT = jnp.transpose(time_feats, (2, 1, 0))

    pa_s, pr_s = proj_W[0:64], proj_W[64:128]
    pn_s, pt_s = proj_W[128:192], proj_W[192:256]
    qnT = (num_W2 @ pn_s).T
    qtT = (time_W2 @ pt_s).T
    bias = (num_b2 @ pn_s + time_b2 @ pt_s + proj_b).reshape(_D, 1)

    d = _tc_dense(nfT, tfT, num_W1.T, num_b1.reshape(_D, 1),
                  time_W1.T, time_b1.reshape(_D, 1), qnT, qtT, bias)
    ap = a.reshape(_N // 2, 128)
    rp = r.reshape(_N // 2, 128)
    outT = _tc_combine(ap, rp, d, pa_s, pr_s)
    return jnp.transpose(outT, (2, 0, 1))
